# baseline (speedup 1.0000x reference)
import functools

import jax
import jax.numpy as jnp
from jax import lax
from jax.experimental import pallas as pl
from jax.experimental.pallas import tpu as pltpu
from jax.experimental.pallas import tpu_sc as plsc

_NC = 2
_NS = 16
_NW = _NC * _NS
_LANES = 16
_C = 16384

_UNIT = 524288
_SC_UNITS = 4
_TOTAL_UNITS = 16


def _sc_partial_sums(n_sc):
  assert n_sc % (_NW * _C) == 0
  per_worker = n_sc // _NW
  num_chunks = per_worker // _C
  mesh = plsc.VectorSubcoreMesh(core_axis_name="c", subcore_axis_name="s")

  @functools.partial(
      pl.kernel,
      out_type=jax.ShapeDtypeStruct((_NW, 2 * _LANES), jnp.float32),
      mesh=mesh,
      scratch_types=[
          pltpu.VMEM((2, _C), jnp.float32),
          pltpu.VMEM((2, _C), jnp.float32),
          pltpu.VMEM((2, _C), jnp.int32),
          pltpu.VMEM((2 * _LANES,), jnp.float32),
          pltpu.SemaphoreType.DMA,
          pltpu.SemaphoreType.DMA,
      ],
  )
  def k(dep_hbm, pred_hbm, gt_hbm, mask_hbm, out_hbm, pbuf, gbuf, mbuf, obuf,
        sem0, sem1):
    del dep_hbm
    wid = lax.axis_index("s") * _NC + lax.axis_index("c")
    base = wid * per_worker
    sems = (sem0, sem1)

    def fire(j, b):
      off = base + j * _C
      return [
          pltpu.async_copy(pred_hbm.at[pl.ds(off, _C)], pbuf.at[b], sems[b]),
          pltpu.async_copy(gt_hbm.at[pl.ds(off, _C)], gbuf.at[b], sems[b]),
          pltpu.async_copy(mask_hbm.at[pl.ds(off, _C)], mbuf.at[b], sems[b]),
      ]

    acc_s = jnp.zeros((_LANES,), jnp.float32)
    acc_c = jnp.zeros((_LANES,), jnp.float32)
    handles = [fire(0, 0), None]
    for j in range(num_chunks):
      b = j & 1
      if j + 1 < num_chunks:
        handles[1 - b] = fire(j + 1, 1 - b)
      for h in handles[b]:
        h.wait()

      def body(i, carry, b=b):
        s, c = carry
        off = i * _LANES
        p = pbuf[b, pl.ds(off, _LANES)]
        g = gbuf[b, pl.ds(off, _LANES)]
        m = mbuf[b, pl.ds(off, _LANES)]
        sel = m > 0
        s = s + jnp.where(sel, jnp.abs(p - g), 0.0)
        c = c + jnp.where(sel, 1.0, 0.0)
        return (s, c)

      acc_s, acc_c = lax.fori_loop(0, _C // _LANES, body, (acc_s, acc_c),
                                   unroll=8)

    obuf[pl.ds(0, _LANES)] = acc_s
    obuf[pl.ds(_LANES, _LANES)] = acc_c
    pltpu.sync_copy(obuf, out_hbm.at[wid])

  return k


_TC_ROWS = 4096


def _tc_body(p_ref, g_ref, m_ref, s_ref, c_ref, acc_ref):
  i = pl.program_id(0)

  @pl.when(i == 0)
  def _():
    acc_ref[...] = jnp.zeros_like(acc_ref)

  sel = m_ref[...] > 0
  d = jnp.where(sel, jnp.abs(p_ref[...] - g_ref[...]), 0.0)
  acc_ref[0:1, :] += jnp.sum(d, axis=0, keepdims=True)
  acc_ref[1:2, :] += jnp.sum(jnp.where(sel, 1.0, 0.0), axis=0,
                             keepdims=True)

  @pl.when(i == pl.num_programs(0) - 1)
  def _():
    s_ref[0] = jnp.sum(acc_ref[0:1, :])
    c_ref[0] = jnp.sum(acc_ref[1:2, :])


def _tc_partial_sums(row_offset_blocks, grid):
  in_spec = pl.BlockSpec((_TC_ROWS, 128),
                         lambda i: (i + row_offset_blocks, 0))
  out_spec = pl.BlockSpec(memory_space=pltpu.SMEM)
  return pl.pallas_call(
      _tc_body,
      grid=(grid,),
      in_specs=[in_spec, in_spec, in_spec],
      out_specs=[out_spec, out_spec],
      out_shape=[
          jax.ShapeDtypeStruct((1,), jnp.float32),
          jax.ShapeDtypeStruct((1,), jnp.float32),
      ],
      scratch_shapes=[pltpu.VMEM((2, 128), jnp.float32)],
  )


_TC_A_UNITS = 3


def kernel(predictions, gt_dose, possible_dose_mask):
  n = predictions.size
  assert n == _TOTAL_UNITS * _UNIT
  p = predictions.reshape(n)
  g = gt_dose.reshape(n)
  m = possible_dose_mask.reshape(n)
  n_sc = _SC_UNITS * _UNIT

  p2 = p.reshape(-1, 128)
  g2 = g.reshape(-1, 128)
  m2 = m.reshape(-1, 128)
  rows_per_unit = _UNIT // 128
  assert rows_per_unit % _TC_ROWS == 0 or _TC_ROWS % rows_per_unit == 0
  blocks_per_unit = rows_per_unit // _TC_ROWS
  tc_b_units = _TOTAL_UNITS - _SC_UNITS - _TC_A_UNITS

  tca_s, tca_c = _tc_partial_sums(
      (_SC_UNITS + tc_b_units) * blocks_per_unit,
      _TC_A_UNITS * blocks_per_unit,
  )(p2, g2, m2)

  parts = _sc_partial_sums(n_sc)(tca_s, p, g, m)

  tcb_s, tcb_c = _tc_partial_sums(
      _SC_UNITS * blocks_per_unit,
      tc_b_units * blocks_per_unit,
  )(p2, g2, m2)

  sc_sums = jnp.sum(parts.reshape(_NW, 2, _LANES), axis=(0, 2))
  total = sc_sums[0] + tca_s[0] + tcb_s[0]
  count = sc_sums[1] + tca_c[0] + tcb_c[0]
  return total / jnp.maximum(count, 1.0)


# speedup vs baseline: 1.0677x; 1.0677x over previous
import functools

import jax
import jax.numpy as jnp
from jax import lax
from jax.experimental import pallas as pl
from jax.experimental.pallas import tpu as pltpu
from jax.experimental.pallas import tpu_sc as plsc

_NC = 2
_NS = 16
_NW = _NC * _NS
_LANES = 16
_C = 16384

_UNIT = 524288
_SC_UNITS = 3
_TOTAL_UNITS = 16


def _sc_partial_sums(n_sc):
  assert n_sc % (_NW * _C) == 0
  per_worker = n_sc // _NW
  num_chunks = per_worker // _C
  mesh = plsc.VectorSubcoreMesh(core_axis_name="c", subcore_axis_name="s")

  @functools.partial(
      pl.kernel,
      out_type=jax.ShapeDtypeStruct((_NW, 2 * _LANES), jnp.float32),
      mesh=mesh,
      scratch_types=[
          pltpu.VMEM((2, _C), jnp.float32),
          pltpu.VMEM((2, _C), jnp.float32),
          pltpu.VMEM((2, _C), jnp.int32),
          pltpu.VMEM((2 * _LANES,), jnp.float32),
          pltpu.SemaphoreType.DMA,
          pltpu.SemaphoreType.DMA,
      ],
  )
  def k(pred_hbm, gt_hbm, mask_hbm, out_hbm, pbuf, gbuf, mbuf, obuf, sem0,
        sem1):
    wid = lax.axis_index("s") * _NC + lax.axis_index("c")
    base = wid * per_worker
    sems = (sem0, sem1)

    def fire(j, b):
      off = base + j * _C
      return [
          pltpu.async_copy(pred_hbm.at[pl.ds(off, _C)], pbuf.at[b], sems[b]),
          pltpu.async_copy(gt_hbm.at[pl.ds(off, _C)], gbuf.at[b], sems[b]),
          pltpu.async_copy(mask_hbm.at[pl.ds(off, _C)], mbuf.at[b], sems[b]),
      ]

    acc_s = jnp.zeros((_LANES,), jnp.float32)
    acc_c = jnp.zeros((_LANES,), jnp.float32)
    handles = [fire(0, 0), None]
    for j in range(num_chunks):
      b = j & 1
      if j + 1 < num_chunks:
        handles[1 - b] = fire(j + 1, 1 - b)
      for h in handles[b]:
        h.wait()

      def body(i, carry, b=b):
        s, c = carry
        off = i * _LANES
        p = pbuf[b, pl.ds(off, _LANES)]
        g = gbuf[b, pl.ds(off, _LANES)]
        m = mbuf[b, pl.ds(off, _LANES)]
        sel = m > 0
        s = s + jnp.where(sel, jnp.abs(p - g), 0.0)
        c = c + jnp.where(sel, 1.0, 0.0)
        return (s, c)

      acc_s, acc_c = lax.fori_loop(0, _C // _LANES, body, (acc_s, acc_c),
                                   unroll=8)

    obuf[pl.ds(0, _LANES)] = acc_s
    obuf[pl.ds(_LANES, _LANES)] = acc_c
    pltpu.sync_copy(obuf, out_hbm.at[wid])

  return k


_TC_ROWS = 4096


def _tc_body(p_ref, g_ref, m_ref, s_ref, c_ref, acc_ref):
  i = pl.program_id(0)

  @pl.when(i == 0)
  def _():
    acc_ref[...] = jnp.zeros_like(acc_ref)

  sel = m_ref[...] > 0
  d = jnp.where(sel, jnp.abs(p_ref[...] - g_ref[...]), 0.0)
  acc_ref[0:1, :] += jnp.sum(d, axis=0, keepdims=True)
  acc_ref[1:2, :] += jnp.sum(jnp.where(sel, 1.0, 0.0), axis=0,
                             keepdims=True)

  @pl.when(i == pl.num_programs(0) - 1)
  def _():
    s_ref[0] = jnp.sum(acc_ref[0:1, :])
    c_ref[0] = jnp.sum(acc_ref[1:2, :])


def _tc_partial_sums(row_offset_blocks, grid):
  in_spec = pl.BlockSpec((_TC_ROWS, 128),
                         lambda i: (i + row_offset_blocks, 0))
  out_spec = pl.BlockSpec(memory_space=pltpu.SMEM)
  return pl.pallas_call(
      _tc_body,
      grid=(grid,),
      in_specs=[in_spec, in_spec, in_spec],
      out_specs=[out_spec, out_spec],
      out_shape=[
          jax.ShapeDtypeStruct((1,), jnp.float32),
          jax.ShapeDtypeStruct((1,), jnp.float32),
      ],
      scratch_shapes=[pltpu.VMEM((2, 128), jnp.float32)],
  )


def kernel(predictions, gt_dose, possible_dose_mask):
  n = predictions.size
  assert n == _TOTAL_UNITS * _UNIT
  p = predictions.reshape(n)
  g = gt_dose.reshape(n)
  m = possible_dose_mask.reshape(n)
  n_sc = _SC_UNITS * _UNIT

  parts = _sc_partial_sums(n_sc)(p, g, m)

  p2 = p.reshape(-1, 128)
  g2 = g.reshape(-1, 128)
  m2 = m.reshape(-1, 128)
  rows_per_unit = _UNIT // 128
  blocks_per_unit = rows_per_unit // _TC_ROWS
  tc_s, tc_c = _tc_partial_sums(
      _SC_UNITS * blocks_per_unit,
      (_TOTAL_UNITS - _SC_UNITS) * blocks_per_unit,
  )(p2, g2, m2)

  sc_sums = jnp.sum(parts.reshape(_NW, 2, _LANES), axis=(0, 2))
  total = sc_sums[0] + tc_s[0]
  count = sc_sums[1] + tc_c[0]
  return total / jnp.maximum(count, 1.0)


# speedup vs baseline: 1.0730x; 1.0049x over previous
import functools

import jax
import jax.numpy as jnp
from jax import lax
from jax.experimental import pallas as pl
from jax.experimental.pallas import tpu as pltpu
from jax.experimental.pallas import tpu_sc as plsc

_NC = 2
_NS = 16
_NW = _NC * _NS
_LANES = 16
_C = 16384

_UNIT = 524288
_SC_UNITS = 2
_TOTAL_UNITS = 16


def _sc_partial_sums(n_sc):
  assert n_sc % (_NW * _C) == 0
  per_worker = n_sc // _NW
  num_chunks = per_worker // _C
  mesh = plsc.VectorSubcoreMesh(core_axis_name="c", subcore_axis_name="s")

  @functools.partial(
      pl.kernel,
      out_type=jax.ShapeDtypeStruct((_NW, 2 * _LANES), jnp.float32),
      mesh=mesh,
      scratch_types=[
          pltpu.VMEM((2, _C), jnp.float32),
          pltpu.VMEM((2, _C), jnp.float32),
          pltpu.VMEM((2, _C), jnp.int32),
          pltpu.VMEM((2 * _LANES,), jnp.float32),
          pltpu.SemaphoreType.DMA,
          pltpu.SemaphoreType.DMA,
      ],
  )
  def k(pred_hbm, gt_hbm, mask_hbm, out_hbm, pbuf, gbuf, mbuf, obuf, sem0,
        sem1):
    wid = lax.axis_index("s") * _NC + lax.axis_index("c")
    base = wid * per_worker
    sems = (sem0, sem1)

    def fire(j, b):
      off = base + j * _C
      return [
          pltpu.async_copy(pred_hbm.at[pl.ds(off, _C)], pbuf.at[b], sems[b]),
          pltpu.async_copy(gt_hbm.at[pl.ds(off, _C)], gbuf.at[b], sems[b]),
          pltpu.async_copy(mask_hbm.at[pl.ds(off, _C)], mbuf.at[b], sems[b]),
      ]

    acc_s = jnp.zeros((_LANES,), jnp.float32)
    acc_c = jnp.zeros((_LANES,), jnp.float32)
    handles = [fire(0, 0), None]
    for j in range(num_chunks):
      b = j & 1
      if j + 1 < num_chunks:
        handles[1 - b] = fire(j + 1, 1 - b)
      for h in handles[b]:
        h.wait()

      def body(i, carry, b=b):
        s, c = carry
        off = i * _LANES
        p = pbuf[b, pl.ds(off, _LANES)]
        g = gbuf[b, pl.ds(off, _LANES)]
        m = mbuf[b, pl.ds(off, _LANES)]
        sel = m > 0
        s = s + jnp.where(sel, jnp.abs(p - g), 0.0)
        c = c + jnp.where(sel, 1.0, 0.0)
        return (s, c)

      acc_s, acc_c = lax.fori_loop(0, _C // _LANES, body, (acc_s, acc_c),
                                   unroll=8)

    obuf[pl.ds(0, _LANES)] = acc_s
    obuf[pl.ds(_LANES, _LANES)] = acc_c
    pltpu.sync_copy(obuf, out_hbm.at[wid])

  return k


_TC_ROWS = 4096


def _tc_body(p_ref, g_ref, m_ref, s_ref, c_ref, acc_ref):
  i = pl.program_id(0)

  @pl.when(i == 0)
  def _():
    acc_ref[...] = jnp.zeros_like(acc_ref)

  sel = m_ref[...] > 0
  d = jnp.where(sel, jnp.abs(p_ref[...] - g_ref[...]), 0.0)
  acc_ref[0:1, :] += jnp.sum(d, axis=0, keepdims=True)
  acc_ref[1:2, :] += jnp.sum(jnp.where(sel, 1.0, 0.0), axis=0,
                             keepdims=True)

  @pl.when(i == pl.num_programs(0) - 1)
  def _():
    s_ref[0] = jnp.sum(acc_ref[0:1, :])
    c_ref[0] = jnp.sum(acc_ref[1:2, :])


def _tc_partial_sums(row_offset_blocks, grid):
  in_spec = pl.BlockSpec((_TC_ROWS, 128),
                         lambda i: (i + row_offset_blocks, 0))
  out_spec = pl.BlockSpec(memory_space=pltpu.SMEM)
  return pl.pallas_call(
      _tc_body,
      grid=(grid,),
      in_specs=[in_spec, in_spec, in_spec],
      out_specs=[out_spec, out_spec],
      out_shape=[
          jax.ShapeDtypeStruct((1,), jnp.float32),
          jax.ShapeDtypeStruct((1,), jnp.float32),
      ],
      scratch_shapes=[pltpu.VMEM((2, 128), jnp.float32)],
  )


def kernel(predictions, gt_dose, possible_dose_mask):
  n = predictions.size
  assert n == _TOTAL_UNITS * _UNIT
  p = predictions.reshape(n)
  g = gt_dose.reshape(n)
  m = possible_dose_mask.reshape(n)
  n_sc = _SC_UNITS * _UNIT

  parts = _sc_partial_sums(n_sc)(p, g, m)

  p2 = p.reshape(-1, 128)
  g2 = g.reshape(-1, 128)
  m2 = m.reshape(-1, 128)
  rows_per_unit = _UNIT // 128
  blocks_per_unit = rows_per_unit // _TC_ROWS
  tc_s, tc_c = _tc_partial_sums(
      _SC_UNITS * blocks_per_unit,
      (_TOTAL_UNITS - _SC_UNITS) * blocks_per_unit,
  )(p2, g2, m2)

  sc_sums = jnp.sum(parts.reshape(_NW, 2, _LANES), axis=(0, 2))
  total = sc_sums[0] + tc_s[0]
  count = sc_sums[1] + tc_c[0]
  return total / jnp.maximum(count, 1.0)
